# baseline (speedup 1.0000x reference)
import functools
import jax
import jax.numpy as jnp
from jax import lax
from jax.experimental import pallas as pl
from jax.experimental.pallas import tpu as pltpu
from jax.experimental.pallas import tpu_sc as plsc

D = 768
P = 1024
ALPHA = 16
HID = 2688
N = 8192
NW = 32
RPW = N // NW
RB = 32
L = 16


def _rms(x, w):
    r = jax.lax.rsqrt(jnp.mean(x * x, axis=-1, keepdims=True) + 1e-6)
    return x * r * w


def _body_a(x_ref, rms1_ref, toP_ref, toPb_ref, u_ref):
    h1 = _rms(x_ref[...], rms1_ref[...])
    u_ref[...] = jax.lax.dot_general(
        h1, toP_ref[...], (((1,), (1,)), ((), ())),
        preferred_element_type=jnp.float32) + toPb_ref[...]


def _proj_u(x, rms1_w, to_P_w, to_P_b):
    T = 1024
    return pl.pallas_call(
        _body_a,
        grid=(N // T,),
        in_specs=[
            pl.BlockSpec((T, D), lambda i: (i, 0)),
            pl.BlockSpec((D,), lambda i: (0,)),
            pl.BlockSpec((P, D), lambda i: (0, 0)),
            pl.BlockSpec((P,), lambda i: (0,)),
        ],
        out_specs=pl.BlockSpec((T, P), lambda i: (i, 0)),
        out_shape=jax.ShapeDtypeStruct((N, P), jnp.float32),
    )(x, rms1_w, to_P_w, to_P_b)


def _splat_i32(v):
    return jnp.broadcast_to(v, (L,)).astype(jnp.int32)


def _scan_hist(h_ref, need, total0):
    zf = jnp.zeros((L,), jnp.float32)
    zi = jnp.zeros((L,), jnp.int32)

    def step(i, carry):
        total, bkt, cab, done = carry
        vi = 15 - i
        h = h_ref[pl.ds(vi * L, L)]
        hr = lax.rev(h, (0,))
        s = jnp.cumsum(hr)
        cum = total + s
        crossed = cum >= need
        npc = plsc.all_reduce_population_count(crossed)
        ffs = plsc.all_reduce_ffs(crossed)
        lane = jnp.where(npc > 0, ffs, zi)
        onehot = (lax.iota(jnp.int32, L) == lane).astype(jnp.float32)
        sL = jnp.broadcast_to(jnp.sum(s * onehot), (L,))
        hL = jnp.broadcast_to(jnp.sum(hr * onehot), (L,))
        found = jnp.logical_and(npc > 0, done == 0)
        bkt = jnp.where(found, vi * L + (15 - lane), bkt)
        cab = jnp.where(found, total + sL - hL, cab)
        done = jnp.where(found, done + 1, done)
        total = jnp.broadcast_to(jnp.sum(h), (L,)) + total
        return (total, bkt, cab, done)

    total, bkt, cab, _ = lax.fori_loop(0, 16, step,
                                       (total0, zi, zf, zi))
    return bkt, cab


def _gate_body(u_hbm, t_hbm, buf, h1, h2):
    wid = lax.axis_index("s") * 2 + lax.axis_index("c")
    base = wid * RPW
    ones = jnp.ones((L,), jnp.float32)
    zeros16 = jnp.zeros((L,), jnp.float32)

    def batch_body(b, _):
        row0 = base + b * RB
        pltpu.sync_copy(u_hbm.at[pl.ds(row0, RB)], buf)

        def row_body(r, _):
            for i in range(16):
                h1[pl.ds(i * L, L)] = zeros16
                h2[pl.ds(i * L, L)] = zeros16

            def p1(j, _):
                v = buf[r, pl.ds(j * L, L)]
                bits = lax.bitcast_convert_type(jnp.abs(v), jnp.int32)
                e = lax.shift_right_logical(bits, 23)
                plsc.addupdate_scatter(h1, [e], ones)
                return 0

            lax.fori_loop(0, P // L, p1, 0)
            need1 = jnp.full((L,), float(ALPHA), jnp.float32)
            ebkt, cab1 = _scan_hist(h1, need1, jnp.zeros((L,), jnp.float32))

            def p2(j, _):
                v = buf[r, pl.ds(j * L, L)]
                bits = lax.bitcast_convert_type(jnp.abs(v), jnp.int32)
                e = lax.shift_right_logical(bits, 23)
                m = jnp.bitwise_and(lax.shift_right_logical(bits, 15), 255)
                plsc.addupdate_scatter(h2, [m], ones, mask=(e == ebkt))
                return 0

            lax.fori_loop(0, P // L, p2, 0)
            need2 = jnp.full((L,), float(ALPHA), jnp.float32) - cab1
            mbkt, _ = _scan_hist(h2, need2, jnp.zeros((L,), jnp.float32))

            thrp = ebkt * 256 + mbkt

            def p3(j, _):
                v = buf[r, pl.ds(j * L, L)]
                bits = lax.bitcast_convert_type(jnp.abs(v), jnp.int32)
                pref = lax.shift_right_logical(bits, 15)
                buf[r, pl.ds(j * L, L)] = jnp.where(pref >= thrp, v, 0.0)
                return 0

            lax.fori_loop(0, P // L, p3, 0)
            return 0

        lax.fori_loop(0, RB, row_body, 0)
        pltpu.sync_copy(buf, t_hbm.at[pl.ds(row0, RB)])
        return 0

    lax.fori_loop(0, RPW // RB, batch_body, 0)


def _gate_sc(u):
    mesh = plsc.VectorSubcoreMesh(core_axis_name="c", subcore_axis_name="s")
    return pl.kernel(
        _gate_body,
        mesh=mesh,
        out_type=jax.ShapeDtypeStruct((N, P), jnp.float32),
        scratch_types=[
            pltpu.VMEM((RB, P), jnp.float32),
            pltpu.VMEM((256,), jnp.float32),
            pltpu.VMEM((256,), jnp.float32),
        ],
        compiler_params=pltpu.CompilerParams(needs_layout_passes=False),
    )(u)


def _body_c(x_ref, t_ref, route_ref, fromP_ref, fromPb_ref, rms2_ref,
            w12_ref, w12b_ref, w3_ref, w3b_ref, out_ref, s_ref):
    @pl.when(pl.program_id(0) == 0)
    def _():
        r = route_ref[...]
        m = jnp.max(r, axis=-1, keepdims=True)
        e = jnp.exp(r - m)
        s_ref[...] = e / jnp.sum(e, axis=-1, keepdims=True)

    x = x_ref[...]
    t = t_ref[...]
    routed = jax.lax.dot_general(t, s_ref[...], (((1,), (0,)), ((), ())),
                                 preferred_element_type=jnp.float32)
    xr = x + jax.lax.dot_general(routed, fromP_ref[...],
                                 (((1,), (1,)), ((), ())),
                                 preferred_element_type=jnp.float32)
    xr = xr + fromPb_ref[...]
    h2 = _rms(xr, rms2_ref[...])
    ab = jax.lax.dot_general(h2, w12_ref[...], (((1,), (1,)), ((), ())),
                             preferred_element_type=jnp.float32)
    ab = ab + w12b_ref[...]
    ga = ab[:, :HID]
    gb = ab[:, HID:]
    y = ga * jax.lax.logistic(ga) * gb
    out = xr + jax.lax.dot_general(y, w3_ref[...], (((1,), (1,)), ((), ())),
                                   preferred_element_type=jnp.float32)
    out_ref[...] = out + w3b_ref[...]


def _tail_tc(x, t, route_w, from_P_w, from_P_b, rms2_w, w12_w, w12_b,
             w3_w, w3_b):
    T = 512
    fixed = lambda i: (0, 0)
    fixed1 = lambda i: (0,)
    return pl.pallas_call(
        _body_c,
        grid=(N // T,),
        in_specs=[
            pl.BlockSpec((T, D), lambda i: (i, 0)),
            pl.BlockSpec((T, P), lambda i: (i, 0)),
            pl.BlockSpec((P, P), fixed),
            pl.BlockSpec((D, P), fixed),
            pl.BlockSpec((D,), fixed1),
            pl.BlockSpec((D,), fixed1),
            pl.BlockSpec((2 * HID, D), fixed),
            pl.BlockSpec((2 * HID,), fixed1),
            pl.BlockSpec((D, HID), fixed),
            pl.BlockSpec((D,), fixed1),
        ],
        out_specs=pl.BlockSpec((T, D), lambda i: (i, 0)),
        out_shape=jax.ShapeDtypeStruct((N, D), jnp.float32),
        scratch_shapes=[pltpu.VMEM((P, P), jnp.float32)],
    )(x, t, route_w, from_P_w, from_P_b, rms2_w, w12_w, w12_b, w3_w, w3_b)


def kernel(x, rms1_w, to_P_w, to_P_b, route_w, from_P_w, from_P_b, rms2_w,
           w12_w, w12_b, w3_w, w3_b):
    u = _proj_u(x, rms1_w, to_P_w, to_P_b)
    t = _gate_sc(u)
    return _tail_tc(x, t, route_w, from_P_w, from_P_b, rms2_w, w12_w,
                    w12_b, w3_w, w3_b)


# speedup vs baseline: 2.9513x; 2.9513x over previous
import jax
import jax.numpy as jnp
from jax.experimental import pallas as pl
from jax.experimental.pallas import tpu as pltpu

D = 768
P = 1024
ALPHA = 16
HID = 2688
N_TILE = 512


def _rms(x, w):
    r = jax.lax.rsqrt(jnp.mean(x * x, axis=-1, keepdims=True) + 1e-6)
    return x * r * w


def _body(x_ref, rms1_ref, toP_ref, toPb_ref, route_ref, fromP_ref,
          fromPb_ref, rms2_ref, w12_ref, w12b_ref, w3_ref, w3b_ref,
          out_ref, s_ref):
    @pl.when(pl.program_id(0) == 0)
    def _():
        r = route_ref[...]
        m = jnp.max(r, axis=-1, keepdims=True)
        e = jnp.exp(r - m)
        s_ref[...] = e / jnp.sum(e, axis=-1, keepdims=True)

    x = x_ref[...]
    h1 = _rms(x, rms1_ref[...])
    U = jax.lax.dot_general(h1, toP_ref[...], (((1,), (1,)), ((), ())),
                            preferred_element_type=jnp.float32)
    U = U + toPb_ref[...]

    absU = jnp.abs(U)
    thresh = jnp.max(absU, axis=-1, keepdims=True)
    for _ in range(ALPHA - 1):
        thresh = jnp.max(jnp.where(absU < thresh, absU, -1.0),
                         axis=-1, keepdims=True)
    t = jnp.where(absU >= thresh, U, 0.0)

    routed = jax.lax.dot_general(t, s_ref[...], (((1,), (0,)), ((), ())),
                                 preferred_element_type=jnp.float32)
    xr = x + jax.lax.dot_general(routed, fromP_ref[...],
                                 (((1,), (1,)), ((), ())),
                                 preferred_element_type=jnp.float32)
    xr = xr + fromPb_ref[...]

    h2 = _rms(xr, rms2_ref[...])
    ab = jax.lax.dot_general(h2, w12_ref[...], (((1,), (1,)), ((), ())),
                             preferred_element_type=jnp.float32)
    ab = ab + w12b_ref[...]
    ga = ab[:, :HID]
    gb = ab[:, HID:]
    y = ga * jax.lax.logistic(ga) * gb
    out = xr + jax.lax.dot_general(y, w3_ref[...], (((1,), (1,)), ((), ())),
                                   preferred_element_type=jnp.float32)
    out_ref[...] = out + w3b_ref[...]


def kernel(x, rms1_w, to_P_w, to_P_b, route_w, from_P_w, from_P_b, rms2_w,
           w12_w, w12_b, w3_w, w3_b):
    n = x.shape[0]
    grid = (n // N_TILE,)
    fixed = lambda i: (0, 0)
    fixed1 = lambda i: (0,)
    return pl.pallas_call(
        _body,
        grid=grid,
        in_specs=[
            pl.BlockSpec((N_TILE, D), lambda i: (i, 0)),
            pl.BlockSpec((D,), fixed1),
            pl.BlockSpec((P, D), fixed),
            pl.BlockSpec((P,), fixed1),
            pl.BlockSpec((P, P), fixed),
            pl.BlockSpec((D, P), fixed),
            pl.BlockSpec((D,), fixed1),
            pl.BlockSpec((D,), fixed1),
            pl.BlockSpec((2 * HID, D), fixed),
            pl.BlockSpec((2 * HID,), fixed1),
            pl.BlockSpec((D, HID), fixed),
            pl.BlockSpec((D,), fixed1),
        ],
        out_specs=pl.BlockSpec((N_TILE, D), lambda i: (i, 0)),
        out_shape=jax.ShapeDtypeStruct((n, D), jnp.float32),
        scratch_shapes=[pltpu.VMEM((P, P), jnp.float32)],
    )(x, rms1_w, to_P_w, to_P_b, route_w, from_P_w, from_P_b, rms2_w,
      w12_w, w12_b, w3_w, w3_b)


# speedup vs baseline: 3.2467x; 1.1001x over previous
import jax
import jax.numpy as jnp
from jax.experimental import pallas as pl
from jax.experimental.pallas import tpu as pltpu

D = 768
P = 1024
ALPHA = 16
HID = 2688
N_TILE = 512


def _rms(x, w):
    r = jax.lax.rsqrt(jnp.mean(x * x, axis=-1, keepdims=True) + 1e-6)
    return x * r * w


def _body(x_ref, rms1_ref, toP_ref, toPb_ref, route_ref, fromP_ref,
          fromPb_ref, rms2_ref, w12_ref, w12b_ref, w3_ref, w3b_ref,
          out_ref, s_ref):
    @pl.when(pl.program_id(0) == 0)
    def _():
        r = route_ref[...]
        m = jnp.max(r, axis=-1, keepdims=True)
        e = jnp.exp(r - m)
        s_ref[...] = e / jnp.sum(e, axis=-1, keepdims=True)

    x = x_ref[...]
    h1 = _rms(x, rms1_ref[...])
    U = jax.lax.dot_general(h1, toP_ref[...], (((1,), (1,)), ((), ())),
                            preferred_element_type=jnp.float32)
    U = U + toPb_ref[...]

    absU = jnp.abs(U).astype(jnp.bfloat16)
    neg = jnp.bfloat16(-1.0)
    thresh = jnp.max(absU, axis=-1, keepdims=True)
    for _ in range(ALPHA - 1):
        thresh = jnp.max(jnp.where(absU < thresh, absU, neg),
                         axis=-1, keepdims=True)
    t = jnp.where(absU >= thresh, U, 0.0)

    routed = jax.lax.dot_general(t, s_ref[...], (((1,), (0,)), ((), ())),
                                 preferred_element_type=jnp.float32)
    xr = x + jax.lax.dot_general(routed, fromP_ref[...],
                                 (((1,), (1,)), ((), ())),
                                 preferred_element_type=jnp.float32)
    xr = xr + fromPb_ref[...]

    h2 = _rms(xr, rms2_ref[...])
    ab = jax.lax.dot_general(h2, w12_ref[...], (((1,), (1,)), ((), ())),
                             preferred_element_type=jnp.float32)
    ab = ab + w12b_ref[...]
    ga = ab[:, :HID]
    gb = ab[:, HID:]
    y = ga * jax.lax.logistic(ga) * gb
    out = xr + jax.lax.dot_general(y, w3_ref[...], (((1,), (1,)), ((), ())),
                                   preferred_element_type=jnp.float32)
    out_ref[...] = out + w3b_ref[...]


def kernel(x, rms1_w, to_P_w, to_P_b, route_w, from_P_w, from_P_b, rms2_w,
           w12_w, w12_b, w3_w, w3_b):
    n = x.shape[0]
    grid = (n // N_TILE,)
    fixed = lambda i: (0, 0)
    fixed1 = lambda i: (0,)
    return pl.pallas_call(
        _body,
        grid=grid,
        in_specs=[
            pl.BlockSpec((N_TILE, D), lambda i: (i, 0)),
            pl.BlockSpec((D,), fixed1),
            pl.BlockSpec((P, D), fixed),
            pl.BlockSpec((P,), fixed1),
            pl.BlockSpec((P, P), fixed),
            pl.BlockSpec((D, P), fixed),
            pl.BlockSpec((D,), fixed1),
            pl.BlockSpec((D,), fixed1),
            pl.BlockSpec((2 * HID, D), fixed),
            pl.BlockSpec((2 * HID,), fixed1),
            pl.BlockSpec((D, HID), fixed),
            pl.BlockSpec((D,), fixed1),
        ],
        out_specs=pl.BlockSpec((N_TILE, D), lambda i: (i, 0)),
        out_shape=jax.ShapeDtypeStruct((n, D), jnp.float32),
        scratch_shapes=[pltpu.VMEM((P, P), jnp.float32)],
    )(x, rms1_w, to_P_w, to_P_b, route_w, from_P_w, from_P_b, rms2_w,
      w12_w, w12_b, w3_w, w3_b)
